# baseline (speedup 1.0000x reference)
import jax
import jax.numpy as jnp
from jax import lax
from jax.experimental import pallas as pl
from jax.experimental.pallas import tpu as pltpu
from jax.experimental.pallas import tpu_sc as plsc

B = 1024
L = 1024
N = 16
NC, NS, LANES = 2, 16, 16
NW = NC * NS
RPW = B // NW
NG = RPW // LANES


def _tree_sum(xs):
    while len(xs) > 1:
        xs = [xs[i] + xs[i + 1] for i in range(0, len(xs), 2)]
    return xs[0]


def _sc_body(idx_hbm, eps_hbm, out_hbm, idx_v, eps_v, out_v):
    wid = lax.axis_index("s") * NC + lax.axis_index("c")
    pltpu.sync_copy(idx_hbm.at[wid], idx_v)
    pltpu.sync_copy(eps_hbm, eps_v)

    def _epilogue(mask, ls0, ls1, acc):
        chosen = jnp.where(mask, ls1, ls0)
        m = jnp.maximum(ls0, ls1)
        mn = jnp.minimum(ls0, ls1)
        y = jnp.exp(2.0 * (mn - m))
        z = y / (2.0 + y)
        z2 = z * z
        log1p = 2.0 * z * (1.0 + z2 * (1.0 / 3 + z2 * (1.0 / 5 + z2 * (
            1.0 / 7 + z2 * (1.0 / 9 + z2 * (1.0 / 11))))))
        return acc + (chosen - (m + 0.5 * log1p))

    def step(t, carry):
        acc0, acc1 = carry[0], carry[1]
        Pa = list(carry[2:2 + N])
        Pb = list(carry[2 + N:])
        base = t * RPW
        s0 = idx_v[pl.ds(base, LANES)]
        s1 = idx_v[pl.ds(base + LANES, LANES)]
        m0 = s0 > 0
        m1 = s1 > 0
        E0 = eps_v[pl.ds(t * (2 * N), N)]
        E1 = eps_v[pl.ds(t * (2 * N) + N, N)]
        ua, wa, ub, wb = [], [], [], []
        for n in range(N):
            e0 = E0[n]
            e1 = E1[n]
            u = Pa[n] * e0
            w = Pa[n] * e1
            Pa[n] = jnp.where(m0, w, u)
            ua.append(u)
            wa.append(w)
            u = Pb[n] * e0
            w = Pb[n] * e1
            Pb[n] = jnp.where(m1, w, u)
            ub.append(u)
            wb.append(w)
        acc0 = _epilogue(m0, _tree_sum(ua), _tree_sum(wa), acc0)
        acc1 = _epilogue(m1, _tree_sum(ub), _tree_sum(wb), acc1)
        return (acc0, acc1, *Pa, *Pb)

    ones = jnp.ones((LANES,), jnp.float32)
    zeros = jnp.zeros((LANES,), jnp.float32)
    carry = lax.fori_loop(0, L, step, (zeros, zeros) + (ones,) * (2 * N))
    out_v[pl.ds(0, LANES)] = carry[0]
    out_v[pl.ds(LANES, LANES)] = carry[1]

    pltpu.sync_copy(out_v, out_hbm.at[pl.ds(wid * RPW, RPW)])


def kernel(inputs, eps):
    idx_r = jnp.transpose(inputs).reshape(L, NW, RPW).transpose(1, 0, 2)
    idx_r = idx_r.reshape(NW, L * RPW)
    eps_r = jnp.transpose(eps, (2, 0, 1)).astype(jnp.float32).reshape(L * 2 * N)
    f = pl.kernel(
        _sc_body,
        out_type=jax.ShapeDtypeStruct((B,), jnp.float32),
        mesh=plsc.VectorSubcoreMesh(core_axis_name="c", subcore_axis_name="s"),
        scratch_types=[
            pltpu.VMEM((L * RPW,), jnp.int32),
            pltpu.VMEM((L * 2 * N,), jnp.float32),
            pltpu.VMEM((RPW,), jnp.float32),
        ],
    )
    return f(idx_r, eps_r)


# speedup vs baseline: 2.1433x; 2.1433x over previous
import jax
import jax.numpy as jnp
from jax import lax
from jax.experimental import pallas as pl
from jax.experimental.pallas import tpu as pltpu
from jax.experimental.pallas import tpu_sc as plsc

B = 1024
L = 1024
N = 16
NC, NS, LANES = 2, 16, 16
NW = NC * NS
SPLIT = 512
RPW = SPLIT // NW
NG = RPW // LANES
TBR = 128
TNB = (B - SPLIT) // TBR


def _tree_sum(xs):
    while len(xs) > 1:
        xs = [xs[i] + xs[i + 1] for i in range(0, len(xs), 2)]
    return xs[0]


def _sc_body(idx_hbm, eps_hbm, out_hbm, idx_v, eps_v, out_v):
    wid = lax.axis_index("s") * NC + lax.axis_index("c")
    pltpu.sync_copy(idx_hbm.at[wid], idx_v)
    pltpu.sync_copy(eps_hbm, eps_v)

    def _epilogue(mask, ls0, ls1, acc):
        chosen = jnp.where(mask, ls1, ls0)
        m = jnp.maximum(ls0, ls1)
        mn = jnp.minimum(ls0, ls1)
        y = jnp.exp(2.0 * (mn - m))
        z = y / (2.0 + y)
        z2 = z * z
        log1p = 2.0 * z * (1.0 + z2 * (1.0 / 3 + z2 * (1.0 / 5 + z2 * (
            1.0 / 7 + z2 * (1.0 / 9 + z2 * (1.0 / 11))))))
        return acc + (chosen - (m + 0.5 * log1p))

    for g in range(NG):
        def step(t, carry, g=g):
            acc = carry[0]
            P = list(carry[1:])
            srow = idx_v[pl.ds(t * RPW + g * LANES, LANES)]
            mask = srow > 0
            E0 = eps_v[pl.ds(t * (2 * N), N)]
            E1 = eps_v[pl.ds(t * (2 * N) + N, N)]
            us, ws = [], []
            for n in range(N):
                e0 = E0[n]
                e1 = E1[n]
                u = P[n] * e0
                w = P[n] * e1
                P[n] = jnp.where(mask, w, u)
                us.append(u)
                ws.append(w)
            acc = _epilogue(mask, _tree_sum(us), _tree_sum(ws), acc)
            return (acc, *P)

        ones = jnp.ones((LANES,), jnp.float32)
        zeros = jnp.zeros((LANES,), jnp.float32)
        carry = lax.fori_loop(0, L, step, (zeros,) + (ones,) * N)
        out_v[pl.ds(g * LANES, LANES)] = carry[0]

    pltpu.sync_copy(out_v, out_hbm.at[pl.ds(wid * RPW, RPW)])


def _sc_call(idx_r, eps_r):
    f = pl.kernel(
        _sc_body,
        out_type=jax.ShapeDtypeStruct((SPLIT,), jnp.float32),
        mesh=plsc.VectorSubcoreMesh(core_axis_name="c", subcore_axis_name="s"),
        scratch_types=[
            pltpu.VMEM((L * RPW,), jnp.int32),
            pltpu.VMEM((L * 2 * N,), jnp.float32),
            pltpu.VMEM((RPW,), jnp.float32),
        ],
    )
    return f(idx_r, eps_r)


def _tc_body(idx_ref, e0_ref, e1_ref, out_ref):
    is1 = idx_ref[...] > 0
    ls0 = jnp.zeros((TBR, L), jnp.float32)
    ls1 = jnp.zeros((TBR, L), jnp.float32)
    for n in range(N):
        e0 = e0_ref[n, :].reshape(1, L)
        e1 = e1_ref[n, :].reshape(1, L)
        x = jnp.where(is1, e1, e0)
        x = jnp.concatenate(
            [jnp.ones((TBR, 1), jnp.float32), x[:, :L - 1]], axis=1)
        d = 1
        while d < L:
            xs = jnp.concatenate(
                [jnp.ones((TBR, d), jnp.float32), x[:, :L - d]], axis=1)
            x = x * xs
            d *= 2
        ls0 = ls0 + x * e0
        ls1 = ls1 + x * e1
    m = jnp.maximum(ls0, ls1)
    mn = jnp.minimum(ls0, ls1)
    lse = m + 0.5 * jnp.log(1.0 + jnp.exp(2.0 * (mn - m)))
    chosen = jnp.where(is1, ls1, ls0)
    out_ref[0, 0, :] = jnp.sum(chosen - lse, axis=1)


def _tc_call(idx_tc, eps):
    f = pl.pallas_call(
        _tc_body,
        grid=(TNB,),
        in_specs=[
            pl.BlockSpec((TBR, L), lambda i: (i, 0)),
            pl.BlockSpec((N, L), lambda i: (0, 0)),
            pl.BlockSpec((N, L), lambda i: (0, 0)),
        ],
        out_specs=pl.BlockSpec((1, 1, TBR), lambda i: (i, 0, 0)),
        out_shape=jax.ShapeDtypeStruct((TNB, 1, TBR), jnp.float32),
    )
    out = f(idx_tc, eps[0], eps[1])
    return out.reshape(B - SPLIT)


def kernel(inputs, eps):
    idx_sc = inputs[:SPLIT]
    idx_r = jnp.transpose(idx_sc).reshape(L, NW, RPW).transpose(1, 0, 2)
    idx_r = idx_r.reshape(NW, L * RPW)
    eps_r = jnp.transpose(eps, (2, 0, 1)).astype(jnp.float32).reshape(L * 2 * N)
    sc_out = _sc_call(idx_r, eps_r)
    tc_out = _tc_call(inputs[SPLIT:], eps)
    return jnp.concatenate([sc_out, tc_out])
